# baseline (speedup 1.0000x reference)
import functools

import jax
import jax.numpy as jnp
from jax.experimental import pallas as pl
from jax.experimental.pallas import tpu as pltpu

_C = 64
_G = 8


def _split(a):
    hi = jax.lax.bitcast_convert_type(
        jax.lax.bitcast_convert_type(a, jnp.uint32) & jnp.uint32(0xFFFF0000),
        jnp.float32)
    return hi, a - hi


_NN = (((1,), (0,)), ((), ()))
_NT = (((1,), (1,)), ((), ()))
_TN = (((0,), (0,)), ((), ()))


def _dot3s(a2, b2, dims):
    ah, al = a2
    bh, bl = b2

    def d(x, y):
        return jax.lax.dot_general(x, y, dims,
                                   preferred_element_type=jnp.float32)

    return d(ah, bh) + d(ah, bl) + d(al, bh)


def _dot3(a, b, dims):
    return _dot3s(_split(a), _split(b), dims)


def _kda_kernel(q_ref, k_ref, v_ref, g_ref, b_ref, o_ref, *s_refs):
    j = pl.program_id(1)
    qs, ks, vs, gs, bs, os_ = (r.at[0] for r in
                               (q_ref, k_ref, v_ref, g_ref, b_ref, o_ref))

    @pl.when(j == 0)
    def _():
        for s_ref in s_refs:
            s_ref[...] = jnp.zeros_like(s_ref)

    grp = len(s_refs)
    c = q_ref.shape[1]
    scale = q_ref.shape[3] ** -0.5
    lanes = range(grp)

    row = jax.lax.broadcasted_iota(jnp.int32, (c, c), 0)
    col = jax.lax.broadcasted_iota(jnp.int32, (c, c), 1)
    tril_inc = (col <= row).astype(jnp.float32)
    strict = (col < row).astype(jnp.float32)
    eye = (col == row).astype(jnp.float32)

    beta = [bs[:, gi, :] for gi in lanes]
    v = [vs[:, gi, :] for gi in lanes]
    s0 = [s_refs[gi][...] for gi in lanes]

    lam, lam_inv, lam_tot = [], [], []
    for gi in lanes:
        gh, gl = _split(gs[:, gi, :])
        lg = (jax.lax.dot(tril_inc, gh, preferred_element_type=jnp.float32)
              + jax.lax.dot(tril_inc, gl, preferred_element_type=jnp.float32))
        lam.append(jnp.exp(lg))
        lam_inv.append(jnp.exp(-lg))
        lam_tot.append(lam[gi][c - 1])

    kq = [jnp.concatenate(
        [ks[:, gi, :] * (beta[gi] * lam[gi]),
         qs[:, gi, :] * (lam[gi] * scale)], 0)
        for gi in lanes]
    ki = [ks[:, gi, :] * lam_inv[gi] for gi in lanes]

    kq2 = [_split(kq[gi]) for gi in lanes]
    ki2 = [_split(ki[gi]) for gi in lanes]

    a2 = [_dot3s(kq2[gi], ki2[gi], _NT) for gi in lanes]
    sv = [_dot3s(kq2[gi], _split(s0[gi]), _NN) for gi in lanes]

    n = [-a2[gi][:c] * strict for gi in lanes]
    p = [eye + n[gi] for gi in lanes]
    n2 = [_split(n[gi]) for gi in lanes]
    for _ in range(5):
        n = [_dot3s(n2[gi], n2[gi], _NN) for gi in lanes]
        n2 = [_split(n[gi]) for gi in lanes]
        p2 = [_split(p[gi]) for gi in lanes]
        p = [p[gi] + _dot3s(p2[gi], n2[gi], _NN) for gi in lanes]

    rhs = [beta[gi] * v[gi] - sv[gi][:c] for gi in lanes]
    u = [_dot3(p[gi], rhs[gi], _NN) for gi in lanes]
    u2 = [_split(u[gi]) for gi in lanes]

    aq = [a2[gi][c:] * tril_inc for gi in lanes]
    for gi in lanes:
        os_[:, gi, :] = sv[gi][c:] + _dot3(aq[gi], u[gi], _NN)

    for gi in lanes:
        s_refs[gi][...] = lam_tot[gi][:, None] * (
            s0[gi] + _dot3s(ki2[gi], u2[gi], _TN))


@functools.partial(jax.jit, static_argnames=("interpret",))
def _run(q, k, v, g, beta, interpret=False):
    B, T, H, K = q.shape
    V = v.shape[-1]
    nc = T // _C
    grp = H

    spec_k = pl.BlockSpec((1, _C, H, K), lambda b, j: (b, j, 0, 0))
    spec_v = pl.BlockSpec((1, _C, H, V), lambda b, j: (b, j, 0, 0))
    spec_b = pl.BlockSpec((1, _C, H, 1), lambda b, j: (b, j, 0, 0))

    out = pl.pallas_call(
        _kda_kernel,
        out_shape=jax.ShapeDtypeStruct((B, T, H, V), jnp.float32),
        grid=(B, nc),
        in_specs=[spec_k, spec_k, spec_v, spec_k, spec_b],
        out_specs=spec_v,
        scratch_shapes=[pltpu.VMEM((K, V), jnp.float32) for _ in range(grp)],
        compiler_params=pltpu.CompilerParams(
            dimension_semantics=("parallel", "arbitrary"),
        ),
        name="kda_chunked",
        interpret=interpret,
    )(q, k, v, g, beta[..., None])

    return out


def kernel(q, k, v, g, beta):
    return _run(q, k, v, g, beta)


# speedup vs baseline: 1.2719x; 1.2719x over previous
import functools

import jax
import jax.numpy as jnp
from jax.experimental import pallas as pl
from jax.experimental.pallas import tpu as pltpu

_C = 64
_G = 16


def _split(a):
    hi = jax.lax.bitcast_convert_type(
        jax.lax.bitcast_convert_type(a, jnp.uint32) & jnp.uint32(0xFFFF0000),
        jnp.float32)
    return hi, a - hi


_NN = (((1,), (0,)), ((), ()))
_NT = (((1,), (1,)), ((), ()))
_TN = (((0,), (0,)), ((), ()))


def _dot3s(a2, b2, dims):
    ah, al = a2
    bh, bl = b2

    def d(x, y):
        return jax.lax.dot_general(x, y, dims,
                                   preferred_element_type=jnp.float32)

    return d(ah, bh) + d(ah, bl) + d(al, bh)


def _dot3(a, b, dims):
    return _dot3s(_split(a), _split(b), dims)


def _kda_kernel(q_ref, k_ref, v_ref, g_ref, b_ref, o_ref, *s_refs):
    j = pl.program_id(1)


    @pl.when(j == 0)
    def _():
        for s_ref in s_refs:
            s_ref[...] = jnp.zeros_like(s_ref)

    grp = len(s_refs)
    c = q_ref.shape[1]
    scale = q_ref.shape[2] ** -0.5
    lanes = range(grp)

    row = jax.lax.broadcasted_iota(jnp.int32, (c, c), 0)
    col = jax.lax.broadcasted_iota(jnp.int32, (c, c), 1)
    tril_inc = (col <= row).astype(jnp.float32)
    strict = (col < row).astype(jnp.float32)
    eye = (col == row).astype(jnp.float32)

    beta = [b_ref[gi] for gi in lanes]
    v = [v_ref[gi] for gi in lanes]
    s0 = [s_refs[gi][...] for gi in lanes]

    lam, lam_inv, lam_tot = [], [], []
    for gi in lanes:
        gh, gl = _split(g_ref[gi])
        lg = (jax.lax.dot(tril_inc, gh, preferred_element_type=jnp.float32)
              + jax.lax.dot(tril_inc, gl, preferred_element_type=jnp.float32))
        lam.append(jnp.exp(lg))
        lam_inv.append(jnp.exp(-lg))
        lam_tot.append(lam[gi][c - 1])

    kq = [jnp.concatenate(
        [k_ref[gi] * (beta[gi] * lam[gi]),
         q_ref[gi] * (lam[gi] * scale)], 0)
        for gi in lanes]
    ki = [k_ref[gi] * lam_inv[gi] for gi in lanes]

    kq2 = [_split(kq[gi]) for gi in lanes]
    ki2 = [_split(ki[gi]) for gi in lanes]

    a2 = [_dot3s(kq2[gi], ki2[gi], _NT) for gi in lanes]
    sv = [_dot3s(kq2[gi], _split(s0[gi]), _NN) for gi in lanes]

    n = [-a2[gi][:c] * strict for gi in lanes]
    p = [eye + n[gi] for gi in lanes]
    n2 = [_split(n[gi]) for gi in lanes]
    for _ in range(5):
        n = [_dot3s(n2[gi], n2[gi], _NN) for gi in lanes]
        n2 = [_split(n[gi]) for gi in lanes]
        p2 = [_split(p[gi]) for gi in lanes]
        p = [p[gi] + _dot3s(p2[gi], n2[gi], _NN) for gi in lanes]

    rhs = [beta[gi] * v[gi] - sv[gi][:c] for gi in lanes]
    u = [_dot3(p[gi], rhs[gi], _NN) for gi in lanes]
    u2 = [_split(u[gi]) for gi in lanes]

    aq = [a2[gi][c:] * tril_inc for gi in lanes]
    for gi in lanes:
        o_ref[gi] = sv[gi][c:] + _dot3(aq[gi], u[gi], _NN)

    for gi in lanes:
        s_refs[gi][...] = lam_tot[gi][:, None] * (
            s0[gi] + _dot3s(ki2[gi], u2[gi], _TN))


@functools.partial(jax.jit, static_argnames=("interpret",))
def _run(q, k, v, g, beta, interpret=False):
    B, T, H, K = q.shape
    V = v.shape[-1]
    BH = B * H
    nc = T // _C

    def to_bh(x):
        return jnp.transpose(x, (0, 2, 1, 3)).reshape(BH, T, x.shape[-1])

    qb = to_bh(q)
    kb = to_bh(k)
    vb = to_bh(v)
    gb = to_bh(g)
    bb = jnp.transpose(beta, (0, 2, 1)).reshape(BH, T, 1)

    grp = _G
    while BH % grp:
        grp //= 2

    spec_k = pl.BlockSpec((grp, _C, K), lambda i, j: (i, j, 0))
    spec_v = pl.BlockSpec((grp, _C, V), lambda i, j: (i, j, 0))
    spec_b = pl.BlockSpec((grp, _C, 1), lambda i, j: (i, j, 0))

    out = pl.pallas_call(
        _kda_kernel,
        out_shape=jax.ShapeDtypeStruct((BH, T, V), jnp.float32),
        grid=(BH // grp, nc),
        in_specs=[spec_k, spec_k, spec_v, spec_k, spec_b],
        out_specs=spec_v,
        scratch_shapes=[pltpu.VMEM((K, V), jnp.float32) for _ in range(grp)],
        compiler_params=pltpu.CompilerParams(
            dimension_semantics=("parallel", "arbitrary"),
        ),
        name="kda_chunked",
        interpret=interpret,
    )(qb, kb, vb, gb, bb)

    return jnp.transpose(out.reshape(B, H, T, V), (0, 2, 1, 3))


def kernel(q, k, v, g, beta):
    return _run(q, k, v, g, beta)


# speedup vs baseline: 1.2878x; 1.0125x over previous
import functools

import jax
import jax.numpy as jnp
from jax.experimental import pallas as pl
from jax.experimental.pallas import tpu as pltpu

_C = 64
_G = 32


def _split(a):
    hi = jax.lax.bitcast_convert_type(
        jax.lax.bitcast_convert_type(a, jnp.uint32) & jnp.uint32(0xFFFF0000),
        jnp.float32)
    return hi, a - hi


_NN = (((1,), (0,)), ((), ()))
_NT = (((1,), (1,)), ((), ()))
_TN = (((0,), (0,)), ((), ()))


def _dot3s(a2, b2, dims):
    ah, al = a2
    bh, bl = b2

    def d(x, y):
        return jax.lax.dot_general(x, y, dims,
                                   preferred_element_type=jnp.float32)

    return d(ah, bh) + d(ah, bl) + d(al, bh)


def _dot3(a, b, dims):
    return _dot3s(_split(a), _split(b), dims)


def _kda_kernel(q_ref, k_ref, v_ref, g_ref, b_ref, o_ref, *s_refs):
    j = pl.program_id(1)


    @pl.when(j == 0)
    def _():
        for s_ref in s_refs:
            s_ref[...] = jnp.zeros_like(s_ref)

    grp = len(s_refs)
    c = q_ref.shape[1]
    scale = q_ref.shape[2] ** -0.5
    lanes = range(grp)

    row = jax.lax.broadcasted_iota(jnp.int32, (c, c), 0)
    col = jax.lax.broadcasted_iota(jnp.int32, (c, c), 1)
    tril_inc = (col <= row).astype(jnp.float32)
    strict = (col < row).astype(jnp.float32)
    eye = (col == row).astype(jnp.float32)

    beta = [b_ref[gi] for gi in lanes]
    v = [v_ref[gi] for gi in lanes]
    s0 = [s_refs[gi][...] for gi in lanes]

    lam, lam_inv, lam_tot = [], [], []
    for gi in lanes:
        gh, gl = _split(g_ref[gi])
        lg = (jax.lax.dot(tril_inc, gh, preferred_element_type=jnp.float32)
              + jax.lax.dot(tril_inc, gl, preferred_element_type=jnp.float32))
        lam.append(jnp.exp(lg))
        lam_inv.append(jnp.exp(-lg))
        lam_tot.append(lam[gi][c - 1])

    kq = [jnp.concatenate(
        [k_ref[gi] * (beta[gi] * lam[gi]),
         q_ref[gi] * (lam[gi] * scale)], 0)
        for gi in lanes]
    ki = [k_ref[gi] * lam_inv[gi] for gi in lanes]

    kq2 = [_split(kq[gi]) for gi in lanes]
    ki2 = [_split(ki[gi]) for gi in lanes]

    a2 = [_dot3s(kq2[gi], ki2[gi], _NT) for gi in lanes]
    sv = [_dot3s(kq2[gi], _split(s0[gi]), _NN) for gi in lanes]

    n = [-a2[gi][:c] * strict for gi in lanes]
    p = [eye + n[gi] for gi in lanes]
    n2 = [_split(n[gi]) for gi in lanes]
    for _ in range(5):
        n = [_dot3s(n2[gi], n2[gi], _NN) for gi in lanes]
        n2 = [_split(n[gi]) for gi in lanes]
        p2 = [_split(p[gi]) for gi in lanes]
        p = [p[gi] + _dot3s(p2[gi], n2[gi], _NN) for gi in lanes]

    rhs = [beta[gi] * v[gi] - sv[gi][:c] for gi in lanes]
    u = [_dot3(p[gi], rhs[gi], _NN) for gi in lanes]
    u2 = [_split(u[gi]) for gi in lanes]

    aq = [a2[gi][c:] * tril_inc for gi in lanes]
    for gi in lanes:
        o_ref[gi] = sv[gi][c:] + _dot3(aq[gi], u[gi], _NN)

    for gi in lanes:
        s_refs[gi][...] = lam_tot[gi][:, None] * (
            s0[gi] + _dot3s(ki2[gi], u2[gi], _TN))


@functools.partial(jax.jit, static_argnames=("interpret",))
def _run(q, k, v, g, beta, interpret=False):
    B, T, H, K = q.shape
    V = v.shape[-1]
    BH = B * H
    nc = T // _C

    def to_bh(x):
        return jnp.transpose(x, (0, 2, 1, 3)).reshape(BH, T, x.shape[-1])

    qb = to_bh(q)
    kb = to_bh(k)
    vb = to_bh(v)
    gb = to_bh(g)
    bb = jnp.transpose(beta, (0, 2, 1)).reshape(BH, T, 1)

    grp = _G
    while BH % grp:
        grp //= 2

    spec_k = pl.BlockSpec((grp, _C, K), lambda i, j: (i, j, 0))
    spec_v = pl.BlockSpec((grp, _C, V), lambda i, j: (i, j, 0))
    spec_b = pl.BlockSpec((grp, _C, 1), lambda i, j: (i, j, 0))

    out = pl.pallas_call(
        _kda_kernel,
        out_shape=jax.ShapeDtypeStruct((BH, T, V), jnp.float32),
        grid=(BH // grp, nc),
        in_specs=[spec_k, spec_k, spec_v, spec_k, spec_b],
        out_specs=spec_v,
        scratch_shapes=[pltpu.VMEM((K, V), jnp.float32) for _ in range(grp)],
        compiler_params=pltpu.CompilerParams(
            dimension_semantics=("parallel", "arbitrary"),
        ),
        name="kda_chunked",
        interpret=interpret,
    )(qb, kb, vb, gb, bb)

    return jnp.transpose(out.reshape(B, H, T, V), (0, 2, 1, 3))


def kernel(q, k, v, g, beta):
    return _run(q, k, v, g, beta)
